# baseline (speedup 1.0000x reference)
import functools

import jax
import jax.numpy as jnp
from jax import lax
from jax.experimental import pallas as pl
from jax.experimental.pallas import tpu as pltpu
from jax.experimental.pallas import tpu_sc as plsc

_CHUNK = 128


def _linear(x, W, b, relu=False, block_n=2000):
    N, Din = x.shape
    Dout = W.shape[1]
    nb = N // block_n

    def body(x_ref, w_ref, b_ref, o_ref):
        acc = jnp.dot(x_ref[...], w_ref[...], preferred_element_type=jnp.float32)
        acc = acc + b_ref[...]
        o_ref[...] = jnp.maximum(acc, 0.0) if relu else acc

    return pl.pallas_call(
        body,
        grid=(nb,),
        in_specs=[
            pl.BlockSpec((block_n, Din), lambda i: (i, 0)),
            pl.BlockSpec((Din, Dout), lambda i: (0, 0)),
            pl.BlockSpec((1, Dout), lambda i: (0, 0)),
        ],
        out_specs=pl.BlockSpec((block_n, Dout), lambda i: (i, 0)),
        out_shape=jax.ShapeDtypeStruct((N, Dout), jnp.float32),
    )(x, W, b.reshape(1, -1))


def _linear_sc(x, W, b, block_n):
    N, Din = x.shape
    nb = N // block_n

    def body(x_ref, w_ref, b_ref, o_ref):
        o_ref[...] = (
            jnp.dot(x_ref[...], w_ref[0], preferred_element_type=jnp.float32)
            + b_ref[0]
        )

    Ws = jnp.stack([W[:, :32], W[:, 32:]])
    bs = jnp.stack([b[:32].reshape(1, 32), b[32:].reshape(1, 32)])
    return pl.pallas_call(
        body,
        grid=(nb, 2),
        in_specs=[
            pl.BlockSpec((block_n, Din), lambda i, g: (i, 0)),
            pl.BlockSpec((1, Din, 32), lambda i, g: (g, 0, 0)),
            pl.BlockSpec((1, 1, 32), lambda i, g: (g, 0, 0)),
        ],
        out_specs=pl.BlockSpec((block_n, 32), lambda i, g: (g * nb + i, 0)),
        out_shape=jax.ShapeDtypeStruct((2 * N, 32), jnp.float32),
    )(x, Ws, bs)


def _etab_packed(ea4, Wblk, btile, block_n=2000):
    E4 = ea4.shape[0]
    nb = E4 // block_n

    def body(x_ref, w_ref, b_ref, o_ref):
        o_ref[...] = (
            jnp.dot(x_ref[...], w_ref[0], preferred_element_type=jnp.float32)
            + b_ref[0]
        )

    return pl.pallas_call(
        body,
        grid=(nb, 2),
        in_specs=[
            pl.BlockSpec((block_n, 64), lambda i, g: (i, 0)),
            pl.BlockSpec((1, 64, 128), lambda i, g: (g, 0, 0)),
            pl.BlockSpec((1, 1, 128), lambda i, g: (g, 0, 0)),
        ],
        out_specs=pl.BlockSpec((block_n, 128), lambda i, g: (g * nb + i, 0)),
        out_shape=jax.ShapeDtypeStruct((2 * E4, 128), jnp.float32),
    )(ea4, Wblk, btile)


def _add_stats(agg, skip, block_n=2000):
    N, H = skip.shape
    nb = N // block_n

    def body(a0_ref, a1_ref, s_ref, m_ref, st_ref, acc_ref):
        i = pl.program_id(0)
        m = jnp.concatenate([a0_ref[...], a1_ref[...]], axis=1) + s_ref[...]
        m_ref[...] = m

        @pl.when(i == 0)
        def _():
            acc_ref[...] = jnp.zeros_like(acc_ref)

        acc_ref[0:1, :] = acc_ref[0:1, :] + jnp.sum(m, axis=0, keepdims=True)
        acc_ref[1:2, :] = acc_ref[1:2, :] + jnp.sum(m * m, axis=0, keepdims=True)

        @pl.when(i == nb - 1)
        def _():
            mean = acc_ref[0:1, :] / N
            var = acc_ref[1:2, :] / N - mean * mean
            st_ref[...] = jnp.concatenate([mean, var], axis=0)

    return pl.pallas_call(
        body,
        grid=(nb,),
        in_specs=[
            pl.BlockSpec((block_n, 32), lambda i: (i, 0)),
            pl.BlockSpec((block_n, 32), lambda i: (nb + i, 0)),
            pl.BlockSpec((block_n, H), lambda i: (i, 0)),
        ],
        out_specs=[
            pl.BlockSpec((block_n, H), lambda i: (i, 0)),
            pl.BlockSpec((2, H), lambda i: (0, 0)),
        ],
        out_shape=[
            jax.ShapeDtypeStruct((N, H), jnp.float32),
            jax.ShapeDtypeStruct((2, H), jnp.float32),
        ],
        scratch_shapes=[pltpu.VMEM((2, H), jnp.float32)],
    )(agg, agg, skip)


def _bn_relu(x, stats, g, b, block_n=2000):
    N, H = x.shape
    nb = N // block_n

    def body(x_ref, st_ref, g_ref, b_ref, o_ref):
        mean = st_ref[0:1, :]
        var = st_ref[1:2, :]
        y = g_ref[...] * (x_ref[...] - mean) * lax.rsqrt(var + 1e-5) + b_ref[...]
        o_ref[...] = jnp.maximum(y, 0.0)

    return pl.pallas_call(
        body,
        grid=(nb,),
        in_specs=[
            pl.BlockSpec((block_n, H), lambda i: (i, 0)),
            pl.BlockSpec((2, H), lambda i: (0, 0)),
            pl.BlockSpec((1, H), lambda i: (0, 0)),
            pl.BlockSpec((1, H), lambda i: (0, 0)),
        ],
        out_specs=pl.BlockSpec((block_n, H), lambda i: (i, 0)),
        out_shape=jax.ShapeDtypeStruct((N, H), jnp.float32),
    )(x, stats, g.reshape(1, -1), b.reshape(1, -1))


@functools.lru_cache(maxsize=None)
def _make_edge_conv(n_dst, n_src, n_edges):
    CH = 80
    ER = CH // 4
    ERB = ER + 4
    n_sub = 16
    assert n_edges % CH == 0
    n_chunks = n_edges // CH
    base_chunks = n_chunks // n_sub
    n_extra = n_chunks - base_chunks * n_sub
    e_rows = n_edges // 4
    n_pairs = base_chunks // 2
    assert base_chunks >= 5 and n_extra == 0
    rows_tile = -(-n_dst // n_sub)
    rows_tile += (-rows_tile) % 8
    rows_last = n_dst - 15 * rows_tile
    assert 0 < rows_last <= rows_tile and rows_last % 8 == 0
    mesh = plsc.VectorSubcoreMesh(core_axis_name="c", subcore_axis_name="s")

    def _chunks(total):
        off = 0
        while off < total:
            sz = min(CH, total - off)
            yield off, sz
            off += sz

    def body(src_h, dst_h, ktab, qtab, vtab, etab, out,
             sraw0, sraw1, draw0, draw1, sadj0, sadj1, dadj0, dadj1,
             dsc0, dsc1, kb0, kb1, qb0, qb1, vb0, vb1, eb0, eb1,
             acc, si0, si1, sg0, sg1):
        c = lax.axis_index("c")
        s = lax.axis_index("s")

        z16 = jnp.zeros((16,), jnp.float32)

        def zloop(i, carry):
            kb0[i, pl.ds(0, 16)] = z16
            kb0[i, pl.ds(16, 16)] = z16
            return carry

        lax.fori_loop(0, CH, zloop, 0)
        row0 = s * rows_tile

        @pl.when(s < n_sub - 1)
        def _():
            for off, sz in _chunks(rows_tile):
                pltpu.sync_copy(kb0.at[pl.ds(0, sz)],
                                acc.at[pl.ds(row0 + off, sz)])

        @pl.when(s == n_sub - 1)
        def _():
            for off, sz in _chunks(rows_last):
                pltpu.sync_copy(kb0.at[pl.ds(0, sz)],
                                acc.at[pl.ds(row0 + off, sz)])

        plsc.subcore_barrier()

        koff = c * n_dst
        qoff = c * n_src
        base = s * base_chunks
        tile_chunks = base_chunks

        bufs = (
            (sraw0, draw0, sadj0, dadj0, dsc0, kb0, qb0, vb0, eb0, si0, sg0),
            (sraw1, draw1, sadj1, dadj1, dsc1, kb1, qb1, vb1, eb1, si1, sg1),
        )

        def idx_descs(j, P):
            sraw, draw = bufs[P][:2]
            si = bufs[P][9]
            e0 = (base + j) * CH
            return (pltpu.make_async_copy(src_h.at[pl.ds(e0, CH)], sraw, si),
                    pltpu.make_async_copy(dst_h.at[pl.ds(e0, CH)], draw, si))

        def gather_descs(j, P):
            _, _, sadj, dadj, _, kb, qb, vb, eb, _, sg = bufs[P]
            dlt = ((base + j) % 2) * 4
            er0 = c * e_rows + (base + j) * ER - dlt
            return (pltpu.make_async_copy(ktab.at[dadj], kb, sg),
                    pltpu.make_async_copy(qtab.at[sadj], qb, sg),
                    pltpu.make_async_copy(vtab.at[sadj], vb, sg),
                    pltpu.make_async_copy(etab.at[pl.ds(er0, ERB)], eb, sg))

        def adjust(P):
            sraw, draw, sadj, dadj, dsc = bufs[P][:5]
            for r in range(CH // 16):
                sl = pl.ds(16 * r, 16)
                sadj[sl] = sraw[sl] + qoff
                dadj[sl] = draw[sl] + koff
                dsc[sl] = draw[sl]

        def stage_in(j, P):
            for d in idx_descs(j, P):
                d.wait()
            adjust(P)
            for d in gather_descs(j, P):
                d.start()

        def compute_scatter(j, P):
            kb, qb, vb, eb = bufs[P][5:9]
            dsc = bufs[P][4]
            dlt = ((base + j) % 2) * 4

            def rb(i, carry):
                for u in range(8):
                    r = i * 8 + u
                    erow = dlt + i * 2 + u // 4
                    for h in range(2):
                        sl = pl.ds(16 * h, 16)
                        esl = pl.ds(32 * (u % 4) + 16 * h, 16)
                        x = kb[r, sl] + qb[r, sl] + eb[erow, esl]
                        gate = 1.0 / (1.0 + jnp.exp(-x))
                        vb[r, sl] = gate * vb[r, sl]
                return carry

            lax.fori_loop(0, CH // 8, rb, 0)
            pltpu.sync_copy(vb, acc.at[dsc], add=True)

        for d in idx_descs(0, 0):
            d.start()
        for d in idx_descs(1, 1):
            d.start()
        stage_in(0, 0)
        for d in idx_descs(2, 0):
            d.start()

        def pair_body(t, carry):
            j0 = 2 * t
            j1 = j0 + 1

            stage_in(j1, 1)

            @pl.when(j1 + 2 < tile_chunks)
            def _():
                for d in idx_descs(j1 + 2, 1):
                    d.start()

            for d in gather_descs(j0, 0):
                d.wait()
            compute_scatter(j0, 0)

            @pl.when(j0 + 2 < tile_chunks)
            def _():
                stage_in(j0 + 2, 0)

                @pl.when(j0 + 4 < tile_chunks)
                def _():
                    for d in idx_descs(j0 + 4, 0):
                        d.start()

            for d in gather_descs(j1, 1):
                d.wait()
            compute_scatter(j1, 1)
            return carry

        lax.fori_loop(0, n_pairs, pair_body, 0)

        if base_chunks % 2 == 1:
            for d in gather_descs(base_chunks - 1, 0):
                d.wait()
            compute_scatter(base_chunks - 1, 0)

        plsc.subcore_barrier()

        obase = c * n_dst + row0

        @pl.when(s < n_sub - 1)
        def _():
            pltpu.sync_copy(acc.at[pl.ds(row0, rows_tile)],
                            out.at[pl.ds(obase, rows_tile)])

        @pl.when(s == n_sub - 1)
        def _():
            pltpu.sync_copy(acc.at[pl.ds(row0, rows_last)],
                            out.at[pl.ds(obase, rows_last)])

    return pl.kernel(
        body,
        out_type=jax.ShapeDtypeStruct((2 * n_dst, 32), jnp.float32),
        mesh=mesh,
        compiler_params=pltpu.CompilerParams(use_tc_tiling_on_sc=False),
        scratch_types=(
            [pltpu.VMEM((CH,), jnp.int32)] * 10
            + [pltpu.VMEM((CH, 32), jnp.float32)] * 6
            + [pltpu.VMEM((ERB, 128), jnp.float32)] * 2
            + [
                pltpu.VMEM_SHARED((n_dst, 32), jnp.float32),
                pltpu.SemaphoreType.DMA,
                pltpu.SemaphoreType.DMA,
                pltpu.SemaphoreType.DMA,
                pltpu.SemaphoreType.DMA,
            ]
        ),
    )


def _edge_conv(ei, ktab, qtab, vtab, etab, n_dst, n_src, n_edges):
    return _make_edge_conv(n_dst, n_src, n_edges)(
        ei[0], ei[1], ktab, qtab, vtab, etab)


@functools.lru_cache(maxsize=None)
def _make_cls(lp):
    assert lp % _CHUNK == 0
    n_chunks = lp // _CHUNK
    n_w = 32
    max_t = -(-n_chunks // n_w)
    mesh = plsc.VectorSubcoreMesh(core_axis_name="c", subcore_axis_name="s")

    def body(el0, el1, pm, pv, out, i0b, i1b, ab, bb, sem):
        c = lax.axis_index("c")
        s = lax.axis_index("s")
        w = s * 2 + c

        def chunk_body(t, carry):
            ch = w + n_w * t

            @pl.when(ch < n_chunks)
            def _():
                e0 = ch * _CHUNK
                pltpu.sync_copy(el0.at[pl.ds(e0, _CHUNK)], i0b)
                pltpu.sync_copy(el1.at[pl.ds(e0, _CHUNK)], i1b)
                cp1 = pltpu.async_copy(pm.at[i0b], ab, sem)
                cp2 = pltpu.async_copy(pv.at[i1b], bb, sem)
                cp1.wait()
                cp2.wait()

                def row_body(i, rcarry):
                    sl = pl.ds(0, 16)
                    ab[i, sl] = ab[i, sl] + bb[i, sl]
                    return rcarry

                lax.fori_loop(0, _CHUNK, row_body, 0)
                pltpu.sync_copy(ab, out.at[pl.ds(e0, _CHUNK)])

            return carry

        lax.fori_loop(0, max_t, chunk_body, 0)

    return pl.kernel(
        body,
        out_type=jax.ShapeDtypeStruct((lp, 16), jnp.float32),
        mesh=mesh,
        compiler_params=pltpu.CompilerParams(use_tc_tiling_on_sc=False),
        scratch_types=[
            pltpu.VMEM((_CHUNK,), jnp.int32),
            pltpu.VMEM((_CHUNK,), jnp.int32),
            pltpu.VMEM((_CHUNK, 16), jnp.float32),
            pltpu.VMEM((_CHUNK, 16), jnp.float32),
            pltpu.SemaphoreType.DMA,
        ],
    )


def _cls_gather(eli_p, pm, pv):
    return _make_cls(eli_p.shape[1])(eli_p[0], eli_p[1], pm, pv)


def _gconv(x_src, x_dst, ei, ea4, p, pre, bn_g, bn_b):
    n_src = x_src.shape[0]
    n_dst = x_dst.shape[0]
    n_edges = 4 * ea4.shape[0]
    We, be = p[pre + 'We'], p[pre + 'be']
    eye4 = jnp.eye(4, dtype=jnp.float32)
    Wblk = jnp.stack([jnp.kron(eye4, We[:, :32]), jnp.kron(eye4, We[:, 32:])])
    btile = jnp.stack([jnp.tile(be[:32], 4).reshape(1, 128),
                       jnp.tile(be[32:], 4).reshape(1, 128)])
    ktab = _linear_sc(x_dst, p[pre + 'Wk'], p[pre + 'bk'], 2000)
    qtab = _linear_sc(x_src, p[pre + 'Wq'], p[pre + 'bq'], 2000)
    vtab = _linear_sc(x_src, p[pre + 'Wv'], p[pre + 'bv'], 2000)
    etab = _etab_packed(ea4, Wblk, btile)
    skip = _linear(x_dst, p[pre + 'Ws'], p[pre + 'bs'])
    agg = _edge_conv(ei, ktab, qtab, vtab, etab, n_dst, n_src, n_edges)
    m_new, stats = _add_stats(agg, skip)
    return _bn_relu(m_new, stats, bn_g, bn_b)


def kernel(x_mouse, x_virus, edge_index_mv, edge_index_vm,
           edge_attr_mv, edge_attr_vm, edge_label_index, params):
    p = params
    H = p['mouse_lin_W'].shape[1]
    L = edge_label_index.shape[1]

    xm = _linear(x_mouse, p['mouse_lin_W'], p['mouse_lin_b'], relu=True)
    xv = _linear(x_virus, p['virus_lin_W'], p['virus_lin_b'], relu=True)

    ea4_mv = jnp.reshape(edge_attr_mv, (edge_attr_mv.shape[0] // 4, -1))
    ea4_vm = jnp.reshape(edge_attr_vm, (edge_attr_vm.shape[0] // 4, -1))

    for layer in (1, 2):
        xm_new = _gconv(xv, xm, edge_index_vm, ea4_vm, p,
                        'conv%d_vm_' % layer,
                        p['bn%d_mouse_g' % layer], p['bn%d_mouse_b' % layer])
        xv_new = _gconv(xm, xv, edge_index_mv, ea4_mv, p,
                        'conv%d_mv_' % layer,
                        p['bn%d_virus_g' % layer], p['bn%d_virus_b' % layer])
        xm, xv = xm_new, xv_new

    Wt = jnp.pad(p['cls_W'][:H], ((0, 0), (0, 14)))
    Wb = jnp.pad(p['cls_W'][H:], ((0, 0), (0, 14)))
    bt = jnp.pad(p['cls_b'], (0, 14))
    pm = _linear(xm, Wt, bt)
    pv = _linear(xv, Wb, jnp.zeros((16,), jnp.float32))

    lp = -(-L // _CHUNK) * _CHUNK
    eli_p = jnp.pad(edge_label_index, ((0, 0), (0, lp - L)))
    logits_p = _cls_gather(eli_p, pm, pv)
    return logits_p[:L, :2]


# speedup vs baseline: 2.5665x; 2.5665x over previous
import functools

import jax
import jax.numpy as jnp
from jax import lax
from jax.experimental import pallas as pl
from jax.experimental.pallas import tpu as pltpu
from jax.experimental.pallas import tpu_sc as plsc

_CHUNK = 128


def _linear(x, W, b, relu=False, block_n=2000):
    N, Din = x.shape
    Dout = W.shape[1]
    nb = N // block_n

    def body(x_ref, w_ref, b_ref, o_ref):
        acc = jnp.dot(x_ref[...], w_ref[...], preferred_element_type=jnp.float32)
        acc = acc + b_ref[...]
        o_ref[...] = jnp.maximum(acc, 0.0) if relu else acc

    return pl.pallas_call(
        body,
        grid=(nb,),
        in_specs=[
            pl.BlockSpec((block_n, Din), lambda i: (i, 0)),
            pl.BlockSpec((Din, Dout), lambda i: (0, 0)),
            pl.BlockSpec((1, Dout), lambda i: (0, 0)),
        ],
        out_specs=pl.BlockSpec((block_n, Dout), lambda i: (i, 0)),
        out_shape=jax.ShapeDtypeStruct((N, Dout), jnp.float32),
    )(x, W, b.reshape(1, -1))


def _linear_sc(x, W, b, block_n):
    N, Din = x.shape
    nb = N // block_n

    def body(x_ref, w_ref, b_ref, o_ref):
        o_ref[...] = (
            jnp.dot(x_ref[...], w_ref[0], preferred_element_type=jnp.float32)
            + b_ref[0]
        )

    Ws = jnp.stack([W[:, :32], W[:, 32:]])
    bs = jnp.stack([b[:32].reshape(1, 32), b[32:].reshape(1, 32)])
    return pl.pallas_call(
        body,
        grid=(nb, 2),
        in_specs=[
            pl.BlockSpec((block_n, Din), lambda i, g: (i, 0)),
            pl.BlockSpec((1, Din, 32), lambda i, g: (g, 0, 0)),
            pl.BlockSpec((1, 1, 32), lambda i, g: (g, 0, 0)),
        ],
        out_specs=pl.BlockSpec((block_n, 32), lambda i, g: (g * nb + i, 0)),
        out_shape=jax.ShapeDtypeStruct((2 * N, 32), jnp.float32),
    )(x, Ws, bs)


def _etab_packed(ea4, Wblk, btile, block_n=2000):
    E4 = ea4.shape[0]
    nb = E4 // block_n

    def body(x_ref, w_ref, b_ref, o_ref):
        o_ref[...] = (
            jnp.dot(x_ref[...], w_ref[0], preferred_element_type=jnp.float32)
            + b_ref[0]
        )

    return pl.pallas_call(
        body,
        grid=(nb, 2),
        in_specs=[
            pl.BlockSpec((block_n, 64), lambda i, g: (i, 0)),
            pl.BlockSpec((1, 64, 128), lambda i, g: (g, 0, 0)),
            pl.BlockSpec((1, 1, 128), lambda i, g: (g, 0, 0)),
        ],
        out_specs=pl.BlockSpec((block_n, 128), lambda i, g: (g * nb + i, 0)),
        out_shape=jax.ShapeDtypeStruct((2 * E4, 128), jnp.float32),
    )(ea4, Wblk, btile)


def _add_stats(agg, skip, block_n=2000):
    N, H = skip.shape
    nb = N // block_n

    def body(a0_ref, a1_ref, s_ref, m_ref, st_ref, acc_ref):
        i = pl.program_id(0)
        m = jnp.concatenate([a0_ref[...], a1_ref[...]], axis=1) + s_ref[...]
        m_ref[...] = m

        @pl.when(i == 0)
        def _():
            acc_ref[...] = jnp.zeros_like(acc_ref)

        acc_ref[0:1, :] = acc_ref[0:1, :] + jnp.sum(m, axis=0, keepdims=True)
        acc_ref[1:2, :] = acc_ref[1:2, :] + jnp.sum(m * m, axis=0, keepdims=True)

        @pl.when(i == nb - 1)
        def _():
            mean = acc_ref[0:1, :] / N
            var = acc_ref[1:2, :] / N - mean * mean
            st_ref[...] = jnp.concatenate([mean, var], axis=0)

    return pl.pallas_call(
        body,
        grid=(nb,),
        in_specs=[
            pl.BlockSpec((block_n, 32), lambda i: (i, 0)),
            pl.BlockSpec((block_n, 32), lambda i: (nb + i, 0)),
            pl.BlockSpec((block_n, H), lambda i: (i, 0)),
        ],
        out_specs=[
            pl.BlockSpec((block_n, H), lambda i: (i, 0)),
            pl.BlockSpec((2, H), lambda i: (0, 0)),
        ],
        out_shape=[
            jax.ShapeDtypeStruct((N, H), jnp.float32),
            jax.ShapeDtypeStruct((2, H), jnp.float32),
        ],
        scratch_shapes=[pltpu.VMEM((2, H), jnp.float32)],
    )(agg, agg, skip)


def _bn_relu(x, stats, g, b, block_n=2000):
    N, H = x.shape
    nb = N // block_n

    def body(x_ref, st_ref, g_ref, b_ref, o_ref):
        mean = st_ref[0:1, :]
        var = st_ref[1:2, :]
        y = g_ref[...] * (x_ref[...] - mean) * lax.rsqrt(var + 1e-5) + b_ref[...]
        o_ref[...] = jnp.maximum(y, 0.0)

    return pl.pallas_call(
        body,
        grid=(nb,),
        in_specs=[
            pl.BlockSpec((block_n, H), lambda i: (i, 0)),
            pl.BlockSpec((2, H), lambda i: (0, 0)),
            pl.BlockSpec((1, H), lambda i: (0, 0)),
            pl.BlockSpec((1, H), lambda i: (0, 0)),
        ],
        out_specs=pl.BlockSpec((block_n, H), lambda i: (i, 0)),
        out_shape=jax.ShapeDtypeStruct((N, H), jnp.float32),
    )(x, stats, g.reshape(1, -1), b.reshape(1, -1))


@functools.lru_cache(maxsize=None)
def _make_edge_conv(n_dst, n_src, n_edges):
    CH = 80
    n_sub = 16
    assert n_edges % CH == 0
    n_chunks = n_edges // CH
    base_chunks = n_chunks // n_sub
    n_extra = n_chunks - base_chunks * n_sub
    e_rows = n_edges // 4
    n_pairs = base_chunks // 2
    assert base_chunks >= 5 and n_extra == 0
    rows_tile = -(-n_dst // n_sub)
    rows_tile += (-rows_tile) % 8
    rows_last = n_dst - 15 * rows_tile
    assert 0 < rows_last <= rows_tile and rows_last % 8 == 0
    mesh = plsc.VectorSubcoreMesh(core_axis_name="c", subcore_axis_name="s")

    def _chunks(total):
        off = 0
        while off < total:
            sz = min(CH, total - off)
            yield off, sz
            off += sz

    def body(src_h, dst_h, ktab, qtab, vtab, etab, out,
             sraw0, sraw1, draw0, draw1, sadj0, sadj1, dadj0, dadj1,
             dsc0, dsc1, kb0, kb1, qb0, qb1, vb0, vb1, eb0, eb1,
             acc, si0, si1, sg0, sg1):
        c = lax.axis_index("c")
        s = lax.axis_index("s")

        z16 = jnp.zeros((16,), jnp.float32)

        def zloop(i, carry):
            kb0[i, pl.ds(0, 16)] = z16
            kb0[i, pl.ds(16, 16)] = z16
            return carry

        lax.fori_loop(0, CH, zloop, 0)
        row0 = s * rows_tile

        @pl.when(s < n_sub - 1)
        def _():
            for off, sz in _chunks(rows_tile):
                pltpu.sync_copy(kb0.at[pl.ds(0, sz)],
                                acc.at[pl.ds(row0 + off, sz)])

        @pl.when(s == n_sub - 1)
        def _():
            for off, sz in _chunks(rows_last):
                pltpu.sync_copy(kb0.at[pl.ds(0, sz)],
                                acc.at[pl.ds(row0 + off, sz)])

        plsc.subcore_barrier()

        koff = c * n_dst
        qoff = c * n_src
        base = s * base_chunks
        tile_chunks = base_chunks

        bufs = (
            (sraw0, draw0, sadj0, dadj0, dsc0, kb0, qb0, vb0, eb0, si0, sg0),
            (sraw1, draw1, sadj1, dadj1, dsc1, kb1, qb1, vb1, eb1, si1, sg1),
        )

        def idx_descs(j, P):
            sraw, draw = bufs[P][:2]
            si = bufs[P][9]
            e0 = (base + j) * CH
            return (pltpu.make_async_copy(src_h.at[pl.ds(e0, CH)], sraw, si),
                    pltpu.make_async_copy(dst_h.at[pl.ds(e0, CH)], draw, si))

        def gather_descs(j, P):
            _, _, sadj, dadj, _, kb, qb, vb, eb, _, sg = bufs[P]
            e0 = c * n_edges + (base + j) * CH
            return (pltpu.make_async_copy(ktab.at[dadj], kb, sg),
                    pltpu.make_async_copy(qtab.at[sadj], qb, sg),
                    pltpu.make_async_copy(vtab.at[sadj], vb, sg),
                    pltpu.make_async_copy(etab.at[pl.ds(e0, CH)], eb, sg))

        def adjust(P):
            sraw, draw, sadj, dadj, dsc = bufs[P][:5]
            for r in range(CH // 16):
                sl = pl.ds(16 * r, 16)
                sadj[sl] = sraw[sl] + qoff
                dadj[sl] = draw[sl] + koff
                dsc[sl] = draw[sl]

        def stage_in(j, P):
            for d in idx_descs(j, P):
                d.wait()
            adjust(P)
            for d in gather_descs(j, P):
                d.start()

        def compute_scatter(j, P):
            kb, qb, vb, eb = bufs[P][5:9]
            dsc = bufs[P][4]

            def rb(i, carry):
                for u in range(8):
                    r = i * 8 + u
                    for h in range(2):
                        sl = pl.ds(16 * h, 16)
                        x = kb[r, sl] + qb[r, sl] + eb[r, sl]
                        gate = 1.0 / (1.0 + jnp.exp(-x))
                        vb[r, sl] = gate * vb[r, sl]
                return carry

            lax.fori_loop(0, CH // 8, rb, 0)
            pltpu.sync_copy(vb, acc.at[dsc], add=True)

        for d in idx_descs(0, 0):
            d.start()
        for d in idx_descs(1, 1):
            d.start()
        stage_in(0, 0)
        for d in idx_descs(2, 0):
            d.start()

        def pair_body(t, carry):
            j0 = 2 * t
            j1 = j0 + 1

            stage_in(j1, 1)

            @pl.when(j1 + 2 < tile_chunks)
            def _():
                for d in idx_descs(j1 + 2, 1):
                    d.start()

            for d in gather_descs(j0, 0):
                d.wait()
            compute_scatter(j0, 0)

            @pl.when(j0 + 2 < tile_chunks)
            def _():
                stage_in(j0 + 2, 0)

                @pl.when(j0 + 4 < tile_chunks)
                def _():
                    for d in idx_descs(j0 + 4, 0):
                        d.start()

            for d in gather_descs(j1, 1):
                d.wait()
            compute_scatter(j1, 1)
            return carry

        lax.fori_loop(0, n_pairs, pair_body, 0)

        if base_chunks % 2 == 1:
            for d in gather_descs(base_chunks - 1, 0):
                d.wait()
            compute_scatter(base_chunks - 1, 0)

        plsc.subcore_barrier()

        obase = c * n_dst + row0

        @pl.when(s < n_sub - 1)
        def _():
            pltpu.sync_copy(acc.at[pl.ds(row0, rows_tile)],
                            out.at[pl.ds(obase, rows_tile)])

        @pl.when(s == n_sub - 1)
        def _():
            pltpu.sync_copy(acc.at[pl.ds(row0, rows_last)],
                            out.at[pl.ds(obase, rows_last)])

    return pl.kernel(
        body,
        out_type=jax.ShapeDtypeStruct((2 * n_dst, 32), jnp.float32),
        mesh=mesh,
        compiler_params=pltpu.CompilerParams(use_tc_tiling_on_sc=False),
        scratch_types=(
            [pltpu.VMEM((CH,), jnp.int32)] * 10
            + [pltpu.VMEM((CH, 32), jnp.float32)] * 8
            + [
                pltpu.VMEM_SHARED((n_dst, 32), jnp.float32),
                pltpu.SemaphoreType.DMA,
                pltpu.SemaphoreType.DMA,
                pltpu.SemaphoreType.DMA,
                pltpu.SemaphoreType.DMA,
            ]
        ),
    )


def _edge_conv(ei, ktab, qtab, vtab, etab, n_dst, n_src, n_edges):
    return _make_edge_conv(n_dst, n_src, n_edges)(
        ei[0], ei[1], ktab, qtab, vtab, etab)


@functools.lru_cache(maxsize=None)
def _make_cls(lp):
    assert lp % _CHUNK == 0
    n_chunks = lp // _CHUNK
    n_w = 32
    max_t = -(-n_chunks // n_w)
    mesh = plsc.VectorSubcoreMesh(core_axis_name="c", subcore_axis_name="s")

    def body(el0, el1, pm, pv, out, i0b, i1b, ab, bb, sem):
        c = lax.axis_index("c")
        s = lax.axis_index("s")
        w = s * 2 + c

        def chunk_body(t, carry):
            ch = w + n_w * t

            @pl.when(ch < n_chunks)
            def _():
                e0 = ch * _CHUNK
                pltpu.sync_copy(el0.at[pl.ds(e0, _CHUNK)], i0b)
                pltpu.sync_copy(el1.at[pl.ds(e0, _CHUNK)], i1b)
                cp1 = pltpu.async_copy(pm.at[i0b], ab, sem)
                cp2 = pltpu.async_copy(pv.at[i1b], bb, sem)
                cp1.wait()
                cp2.wait()

                def row_body(i, rcarry):
                    sl = pl.ds(0, 16)
                    ab[i, sl] = ab[i, sl] + bb[i, sl]
                    return rcarry

                lax.fori_loop(0, _CHUNK, row_body, 0)
                pltpu.sync_copy(ab, out.at[pl.ds(e0, _CHUNK)])

            return carry

        lax.fori_loop(0, max_t, chunk_body, 0)

    return pl.kernel(
        body,
        out_type=jax.ShapeDtypeStruct((lp, 16), jnp.float32),
        mesh=mesh,
        compiler_params=pltpu.CompilerParams(use_tc_tiling_on_sc=False),
        scratch_types=[
            pltpu.VMEM((_CHUNK,), jnp.int32),
            pltpu.VMEM((_CHUNK,), jnp.int32),
            pltpu.VMEM((_CHUNK, 16), jnp.float32),
            pltpu.VMEM((_CHUNK, 16), jnp.float32),
            pltpu.SemaphoreType.DMA,
        ],
    )


def _cls_gather(eli_p, pm, pv):
    return _make_cls(eli_p.shape[1])(eli_p[0], eli_p[1], pm, pv)


def _gconv(x_src, x_dst, ei, ea4, p, pre, bn_g, bn_b):
    n_src = x_src.shape[0]
    n_dst = x_dst.shape[0]
    n_edges = 4 * ea4.shape[0]
    We, be = p[pre + 'We'], p[pre + 'be']
    eye4 = jnp.eye(4, dtype=jnp.float32)
    Wblk = jnp.stack([jnp.kron(eye4, We[:, :32]), jnp.kron(eye4, We[:, 32:])])
    btile = jnp.stack([jnp.tile(be[:32], 4).reshape(1, 128),
                       jnp.tile(be[32:], 4).reshape(1, 128)])
    ktab = _linear_sc(x_dst, p[pre + 'Wk'], p[pre + 'bk'], 2000)
    qtab = _linear_sc(x_src, p[pre + 'Wq'], p[pre + 'bq'], 2000)
    vtab = _linear_sc(x_src, p[pre + 'Wv'], p[pre + 'bv'], 2000)
    etab = _etab_packed(ea4, Wblk, btile).reshape(2 * n_edges, 32)
    skip = _linear(x_dst, p[pre + 'Ws'], p[pre + 'bs'])
    agg = _edge_conv(ei, ktab, qtab, vtab, etab, n_dst, n_src, n_edges)
    m_new, stats = _add_stats(agg, skip)
    return _bn_relu(m_new, stats, bn_g, bn_b)


def kernel(x_mouse, x_virus, edge_index_mv, edge_index_vm,
           edge_attr_mv, edge_attr_vm, edge_label_index, params):
    p = params
    H = p['mouse_lin_W'].shape[1]
    L = edge_label_index.shape[1]

    xm = _linear(x_mouse, p['mouse_lin_W'], p['mouse_lin_b'], relu=True)
    xv = _linear(x_virus, p['virus_lin_W'], p['virus_lin_b'], relu=True)

    ea4_mv = jnp.reshape(edge_attr_mv, (edge_attr_mv.shape[0] // 4, -1))
    ea4_vm = jnp.reshape(edge_attr_vm, (edge_attr_vm.shape[0] // 4, -1))

    for layer in (1, 2):
        xm_new = _gconv(xv, xm, edge_index_vm, ea4_vm, p,
                        'conv%d_vm_' % layer,
                        p['bn%d_mouse_g' % layer], p['bn%d_mouse_b' % layer])
        xv_new = _gconv(xm, xv, edge_index_mv, ea4_mv, p,
                        'conv%d_mv_' % layer,
                        p['bn%d_virus_g' % layer], p['bn%d_virus_b' % layer])
        xm, xv = xm_new, xv_new

    Wt = jnp.pad(p['cls_W'][:H], ((0, 0), (0, 14)))
    Wb = jnp.pad(p['cls_W'][H:], ((0, 0), (0, 14)))
    bt = jnp.pad(p['cls_b'], (0, 14))
    pm = _linear(xm, Wt, bt)
    pv = _linear(xv, Wb, jnp.zeros((16,), jnp.float32))

    lp = -(-L // _CHUNK) * _CHUNK
    eli_p = jnp.pad(edge_label_index, ((0, 0), (0, lp - L)))
    logits_p = _cls_gather(eli_p, pm, pv)
    return logits_p[:L, :2]


# speedup vs baseline: 2.5669x; 1.0002x over previous
import functools

import jax
import jax.numpy as jnp
from jax import lax
from jax.experimental import pallas as pl
from jax.experimental.pallas import tpu as pltpu
from jax.experimental.pallas import tpu_sc as plsc

_CHUNK = 128


def _linear(x, W, b, relu=False, block_n=2000):
    N, Din = x.shape
    Dout = W.shape[1]
    nb = N // block_n

    def body(x_ref, w_ref, b_ref, o_ref):
        acc = jnp.dot(x_ref[...], w_ref[...], preferred_element_type=jnp.float32)
        acc = acc + b_ref[...]
        o_ref[...] = jnp.maximum(acc, 0.0) if relu else acc

    return pl.pallas_call(
        body,
        grid=(nb,),
        in_specs=[
            pl.BlockSpec((block_n, Din), lambda i: (i, 0)),
            pl.BlockSpec((Din, Dout), lambda i: (0, 0)),
            pl.BlockSpec((1, Dout), lambda i: (0, 0)),
        ],
        out_specs=pl.BlockSpec((block_n, Dout), lambda i: (i, 0)),
        out_shape=jax.ShapeDtypeStruct((N, Dout), jnp.float32),
    )(x, W, b.reshape(1, -1))


def _linear_sc(x, W, b, block_n):
    N, Din = x.shape
    nb = N // block_n

    def body(x_ref, w_ref, b_ref, o_ref):
        o_ref[...] = (
            jnp.dot(x_ref[...], w_ref[0], preferred_element_type=jnp.float32)
            + b_ref[0]
        )

    Ws = jnp.stack([W[:, :32], W[:, 32:]])
    bs = jnp.stack([b[:32].reshape(1, 32), b[32:].reshape(1, 32)])
    return pl.pallas_call(
        body,
        grid=(nb, 2),
        in_specs=[
            pl.BlockSpec((block_n, Din), lambda i, g: (i, 0)),
            pl.BlockSpec((1, Din, 32), lambda i, g: (g, 0, 0)),
            pl.BlockSpec((1, 1, 32), lambda i, g: (g, 0, 0)),
        ],
        out_specs=pl.BlockSpec((block_n, 32), lambda i, g: (g * nb + i, 0)),
        out_shape=jax.ShapeDtypeStruct((2 * N, 32), jnp.float32),
    )(x, Ws, bs)


def _etab_packed(ea, Wblk, btile, block_n=2000):
    E4 = ea.shape[0] // 4
    nb = E4 // block_n
    ea4 = jnp.reshape(ea, (E4, 64))

    def body(x_ref, w_ref, b_ref, o_ref):
        o_ref[...] = (
            jnp.dot(x_ref[...], w_ref[0], preferred_element_type=jnp.float32)
            + b_ref[0]
        )

    return pl.pallas_call(
        body,
        grid=(nb, 2),
        in_specs=[
            pl.BlockSpec((block_n, 64), lambda i, g: (i, 0)),
            pl.BlockSpec((1, 64, 128), lambda i, g: (g, 0, 0)),
            pl.BlockSpec((1, 1, 128), lambda i, g: (g, 0, 0)),
        ],
        out_specs=pl.BlockSpec((block_n, 128), lambda i, g: (g * nb + i, 0)),
        out_shape=jax.ShapeDtypeStruct((2 * E4, 128), jnp.float32),
    )(ea4, Wblk, btile)


def _add_stats(agg, skip, block_n=2000):
    N, H = skip.shape
    nb = N // block_n

    def body(a0_ref, a1_ref, s_ref, m_ref, st_ref, acc_ref):
        i = pl.program_id(0)
        m = jnp.concatenate([a0_ref[...], a1_ref[...]], axis=1) + s_ref[...]
        m_ref[...] = m

        @pl.when(i == 0)
        def _():
            acc_ref[...] = jnp.zeros_like(acc_ref)

        acc_ref[0:1, :] = acc_ref[0:1, :] + jnp.sum(m, axis=0, keepdims=True)
        acc_ref[1:2, :] = acc_ref[1:2, :] + jnp.sum(m * m, axis=0, keepdims=True)

        @pl.when(i == nb - 1)
        def _():
            mean = acc_ref[0:1, :] / N
            var = acc_ref[1:2, :] / N - mean * mean
            st_ref[...] = jnp.concatenate([mean, var], axis=0)

    return pl.pallas_call(
        body,
        grid=(nb,),
        in_specs=[
            pl.BlockSpec((block_n, 32), lambda i: (i, 0)),
            pl.BlockSpec((block_n, 32), lambda i: (nb + i, 0)),
            pl.BlockSpec((block_n, H), lambda i: (i, 0)),
        ],
        out_specs=[
            pl.BlockSpec((block_n, H), lambda i: (i, 0)),
            pl.BlockSpec((2, H), lambda i: (0, 0)),
        ],
        out_shape=[
            jax.ShapeDtypeStruct((N, H), jnp.float32),
            jax.ShapeDtypeStruct((2, H), jnp.float32),
        ],
        scratch_shapes=[pltpu.VMEM((2, H), jnp.float32)],
    )(agg, agg, skip)


def _bn_relu(x, stats, g, b, block_n=2000):
    N, H = x.shape
    nb = N // block_n

    def body(x_ref, st_ref, g_ref, b_ref, o_ref):
        mean = st_ref[0:1, :]
        var = st_ref[1:2, :]
        y = g_ref[...] * (x_ref[...] - mean) * lax.rsqrt(var + 1e-5) + b_ref[...]
        o_ref[...] = jnp.maximum(y, 0.0)

    return pl.pallas_call(
        body,
        grid=(nb,),
        in_specs=[
            pl.BlockSpec((block_n, H), lambda i: (i, 0)),
            pl.BlockSpec((2, H), lambda i: (0, 0)),
            pl.BlockSpec((1, H), lambda i: (0, 0)),
            pl.BlockSpec((1, H), lambda i: (0, 0)),
        ],
        out_specs=pl.BlockSpec((block_n, H), lambda i: (i, 0)),
        out_shape=jax.ShapeDtypeStruct((N, H), jnp.float32),
    )(x, stats, g.reshape(1, -1), b.reshape(1, -1))


@functools.lru_cache(maxsize=None)
def _make_edge_conv(n_dst, n_src, n_edges):
    CH = 80
    n_sub = 16
    assert n_edges % CH == 0
    n_chunks = n_edges // CH
    base_chunks = n_chunks // n_sub
    n_extra = n_chunks - base_chunks * n_sub
    e_rows = n_edges // 4
    n_pairs = base_chunks // 2
    assert base_chunks >= 5 and n_extra == 0
    rows_tile = -(-n_dst // n_sub)
    rows_tile += (-rows_tile) % 8
    rows_last = n_dst - 15 * rows_tile
    assert 0 < rows_last <= rows_tile and rows_last % 8 == 0
    mesh = plsc.VectorSubcoreMesh(core_axis_name="c", subcore_axis_name="s")

    def _chunks(total):
        off = 0
        while off < total:
            sz = min(CH, total - off)
            yield off, sz
            off += sz

    def body(src_h, dst_h, ktab, qtab, vtab, etab, out,
             sraw0, sraw1, draw0, draw1, sadj0, sadj1, dadj0, dadj1,
             dsc0, dsc1, kb0, kb1, qb0, qb1, vb0, vb1, eb0, eb1,
             acc, si0, si1, sg0, sg1):
        c = lax.axis_index("c")
        s = lax.axis_index("s")

        z16 = jnp.zeros((16,), jnp.float32)

        def zloop(i, carry):
            kb0[i, pl.ds(0, 16)] = z16
            kb0[i, pl.ds(16, 16)] = z16
            return carry

        lax.fori_loop(0, CH, zloop, 0)
        row0 = s * rows_tile

        @pl.when(s < n_sub - 1)
        def _():
            for off, sz in _chunks(rows_tile):
                pltpu.sync_copy(kb0.at[pl.ds(0, sz)],
                                acc.at[pl.ds(row0 + off, sz)])

        @pl.when(s == n_sub - 1)
        def _():
            for off, sz in _chunks(rows_last):
                pltpu.sync_copy(kb0.at[pl.ds(0, sz)],
                                acc.at[pl.ds(row0 + off, sz)])

        plsc.subcore_barrier()

        koff = c * n_dst
        qoff = c * n_src
        base = s * base_chunks
        tile_chunks = base_chunks

        bufs = (
            (sraw0, draw0, sadj0, dadj0, dsc0, kb0, qb0, vb0, eb0, si0, sg0),
            (sraw1, draw1, sadj1, dadj1, dsc1, kb1, qb1, vb1, eb1, si1, sg1),
        )

        def idx_descs(j, P):
            sraw, draw = bufs[P][:2]
            si = bufs[P][9]
            e0 = (base + j) * CH
            return (pltpu.make_async_copy(src_h.at[pl.ds(e0, CH)], sraw, si),
                    pltpu.make_async_copy(dst_h.at[pl.ds(e0, CH)], draw, si))

        def gather_descs(j, P):
            _, _, sadj, dadj, _, kb, qb, vb, eb, _, sg = bufs[P]
            e0 = c * n_edges + (base + j) * CH
            return (pltpu.make_async_copy(ktab.at[dadj], kb, sg),
                    pltpu.make_async_copy(qtab.at[sadj], qb, sg),
                    pltpu.make_async_copy(vtab.at[sadj], vb, sg),
                    pltpu.make_async_copy(etab.at[pl.ds(e0, CH)], eb, sg))

        def adjust(P):
            sraw, draw, sadj, dadj, dsc = bufs[P][:5]
            for r in range(CH // 16):
                sl = pl.ds(16 * r, 16)
                sadj[sl] = sraw[sl] + qoff
                dadj[sl] = draw[sl] + koff
                dsc[sl] = draw[sl]

        def stage_in(j, P):
            for d in idx_descs(j, P):
                d.wait()
            adjust(P)
            for d in gather_descs(j, P):
                d.start()

        def compute_scatter(j, P):
            kb, qb, vb, eb = bufs[P][5:9]
            dsc = bufs[P][4]

            def rb(i, carry):
                for u in range(8):
                    r = i * 8 + u
                    for h in range(2):
                        sl = pl.ds(16 * h, 16)
                        x = kb[r, sl] + qb[r, sl] + eb[r, sl]
                        gate = 1.0 / (1.0 + jnp.exp(-x))
                        vb[r, sl] = gate * vb[r, sl]
                return carry

            lax.fori_loop(0, CH // 8, rb, 0)
            pltpu.sync_copy(vb, acc.at[dsc], add=True)

        for d in idx_descs(0, 0):
            d.start()
        for d in idx_descs(1, 1):
            d.start()
        stage_in(0, 0)
        for d in idx_descs(2, 0):
            d.start()

        def pair_body(t, carry):
            j0 = 2 * t
            j1 = j0 + 1

            stage_in(j1, 1)

            @pl.when(j1 + 2 < tile_chunks)
            def _():
                for d in idx_descs(j1 + 2, 1):
                    d.start()

            for d in gather_descs(j0, 0):
                d.wait()
            compute_scatter(j0, 0)

            @pl.when(j0 + 2 < tile_chunks)
            def _():
                stage_in(j0 + 2, 0)

                @pl.when(j0 + 4 < tile_chunks)
                def _():
                    for d in idx_descs(j0 + 4, 0):
                        d.start()

            for d in gather_descs(j1, 1):
                d.wait()
            compute_scatter(j1, 1)
            return carry

        lax.fori_loop(0, n_pairs, pair_body, 0)

        if base_chunks % 2 == 1:
            for d in gather_descs(base_chunks - 1, 0):
                d.wait()
            compute_scatter(base_chunks - 1, 0)

        plsc.subcore_barrier()

        obase = c * n_dst + row0

        @pl.when(s < n_sub - 1)
        def _():
            pltpu.sync_copy(acc.at[pl.ds(row0, rows_tile)],
                            out.at[pl.ds(obase, rows_tile)])

        @pl.when(s == n_sub - 1)
        def _():
            pltpu.sync_copy(acc.at[pl.ds(row0, rows_last)],
                            out.at[pl.ds(obase, rows_last)])

    return pl.kernel(
        body,
        out_type=jax.ShapeDtypeStruct((2 * n_dst, 32), jnp.float32),
        mesh=mesh,
        compiler_params=pltpu.CompilerParams(use_tc_tiling_on_sc=False),
        scratch_types=(
            [pltpu.VMEM((CH,), jnp.int32)] * 10
            + [pltpu.VMEM((CH, 32), jnp.float32)] * 8
            + [
                pltpu.VMEM_SHARED((n_dst, 32), jnp.float32),
                pltpu.SemaphoreType.DMA,
                pltpu.SemaphoreType.DMA,
                pltpu.SemaphoreType.DMA,
                pltpu.SemaphoreType.DMA,
            ]
        ),
    )


def _edge_conv(ei, ktab, qtab, vtab, etab, n_dst, n_src, n_edges):
    return _make_edge_conv(n_dst, n_src, n_edges)(
        ei[0], ei[1], ktab, qtab, vtab, etab)


@functools.lru_cache(maxsize=None)
def _make_cls(lp):
    assert lp % _CHUNK == 0
    n_chunks = lp // _CHUNK
    n_w = 32
    max_t = -(-n_chunks // n_w)
    mesh = plsc.VectorSubcoreMesh(core_axis_name="c", subcore_axis_name="s")

    def body(el0, el1, pm, pv, out, i0b, i1b, ab, bb, sem):
        c = lax.axis_index("c")
        s = lax.axis_index("s")
        w = s * 2 + c

        def chunk_body(t, carry):
            ch = w + n_w * t

            @pl.when(ch < n_chunks)
            def _():
                e0 = ch * _CHUNK
                pltpu.sync_copy(el0.at[pl.ds(e0, _CHUNK)], i0b)
                pltpu.sync_copy(el1.at[pl.ds(e0, _CHUNK)], i1b)
                cp1 = pltpu.async_copy(pm.at[i0b], ab, sem)
                cp2 = pltpu.async_copy(pv.at[i1b], bb, sem)
                cp1.wait()
                cp2.wait()

                def row_body(i, rcarry):
                    sl = pl.ds(0, 16)
                    ab[i, sl] = ab[i, sl] + bb[i, sl]
                    return rcarry

                lax.fori_loop(0, _CHUNK, row_body, 0)
                pltpu.sync_copy(ab, out.at[pl.ds(e0, _CHUNK)])

            return carry

        lax.fori_loop(0, max_t, chunk_body, 0)

    return pl.kernel(
        body,
        out_type=jax.ShapeDtypeStruct((lp, 16), jnp.float32),
        mesh=mesh,
        compiler_params=pltpu.CompilerParams(use_tc_tiling_on_sc=False),
        scratch_types=[
            pltpu.VMEM((_CHUNK,), jnp.int32),
            pltpu.VMEM((_CHUNK,), jnp.int32),
            pltpu.VMEM((_CHUNK, 16), jnp.float32),
            pltpu.VMEM((_CHUNK, 16), jnp.float32),
            pltpu.SemaphoreType.DMA,
        ],
    )


def _cls_gather(eli_p, pm, pv):
    return _make_cls(eli_p.shape[1])(eli_p[0], eli_p[1], pm, pv)


def _gconv(x_src, x_dst, ei, ea, p, pre, bn_g, bn_b):
    n_src = x_src.shape[0]
    n_dst = x_dst.shape[0]
    n_edges = ea.shape[0]
    We, be = p[pre + 'We'], p[pre + 'be']
    eye4 = jnp.eye(4, dtype=jnp.float32)
    Wblk = jnp.stack([jnp.kron(eye4, We[:, :32]), jnp.kron(eye4, We[:, 32:])])
    btile = jnp.stack([jnp.tile(be[:32], 4).reshape(1, 128),
                       jnp.tile(be[32:], 4).reshape(1, 128)])
    ktab = _linear_sc(x_dst, p[pre + 'Wk'], p[pre + 'bk'], 2000)
    qtab = _linear_sc(x_src, p[pre + 'Wq'], p[pre + 'bq'], 2000)
    vtab = _linear_sc(x_src, p[pre + 'Wv'], p[pre + 'bv'], 2000)
    etab = _etab_packed(ea, Wblk, btile).reshape(2 * n_edges, 32)
    skip = _linear(x_dst, p[pre + 'Ws'], p[pre + 'bs'])
    agg = _edge_conv(ei, ktab, qtab, vtab, etab, n_dst, n_src, n_edges)
    m_new, stats = _add_stats(agg, skip)
    return _bn_relu(m_new, stats, bn_g, bn_b)


def kernel(x_mouse, x_virus, edge_index_mv, edge_index_vm,
           edge_attr_mv, edge_attr_vm, edge_label_index, params):
    p = params
    H = p['mouse_lin_W'].shape[1]
    L = edge_label_index.shape[1]

    xm = _linear(x_mouse, p['mouse_lin_W'], p['mouse_lin_b'], relu=True)
    xv = _linear(x_virus, p['virus_lin_W'], p['virus_lin_b'], relu=True)

    for layer in (1, 2):
        xm_new = _gconv(xv, xm, edge_index_vm, edge_attr_vm, p,
                        'conv%d_vm_' % layer,
                        p['bn%d_mouse_g' % layer], p['bn%d_mouse_b' % layer])
        xv_new = _gconv(xm, xv, edge_index_mv, edge_attr_mv, p,
                        'conv%d_mv_' % layer,
                        p['bn%d_virus_g' % layer], p['bn%d_virus_b' % layer])
        xm, xv = xm_new, xv_new

    Wt = jnp.pad(p['cls_W'][:H], ((0, 0), (0, 14)))
    Wb = jnp.pad(p['cls_W'][H:], ((0, 0), (0, 14)))
    bt = jnp.pad(p['cls_b'], (0, 14))
    pm = _linear(xm, Wt, bt)
    pv = _linear(xv, Wb, jnp.zeros((16,), jnp.float32))

    lp = -(-L // _CHUNK) * _CHUNK
    eli_p = jnp.pad(edge_label_index, ((0, 0), (0, lp - L)))
    logits_p = _cls_gather(eli_p, pm, pv)
    return logits_p[:L, :2]
